# baseline (speedup 1.0000x reference)
import jax
import jax.numpy as jnp
from jax.experimental import pallas as pl
from jax.experimental.pallas import tpu as pltpu

_N = 32768
_B = 2048
_NBLK = _N // _B
_NBUF = 4
_OBUF = 2


def _mlp_kernel(
    x_hbm,
    w1_ref,
    b1_ref,
    w2_ref,
    b2_ref,
    w3_ref,
    b3_ref,
    out_hbm,
    xbuf,
    obuf,
    insem,
    outsem,
):
    def in_copy(j):
        return pltpu.make_async_copy(
            x_hbm.at[pl.ds(j * _B, _B), :], xbuf.at[j % _NBUF], insem.at[j % _NBUF]
        )

    def out_copy(j):
        return pltpu.make_async_copy(
            obuf.at[j % _OBUF], out_hbm.at[pl.ds(j * _B, _B), :], outsem.at[j % _OBUF]
        )

    for j in range(_NBUF):
        in_copy(j).start()
    for i in range(_NBLK):
        in_copy(i).wait()
        h = jnp.tanh(
            jnp.dot(xbuf[i % _NBUF], w1_ref[...], preferred_element_type=jnp.float32)
            + b1_ref[...]
        )
        h = jnp.tanh(
            jnp.dot(h, w2_ref[...], preferred_element_type=jnp.float32) + b2_ref[...]
        )
        y = jnp.dot(h, w3_ref[...], preferred_element_type=jnp.float32) + b3_ref[...]
        if i >= _OBUF:
            out_copy(i - _OBUF).wait()
        obuf[i % _OBUF] = y
        out_copy(i).start()
        if i + _NBUF < _NBLK:
            in_copy(i + _NBUF).start()
    for j in range(_NBLK - _OBUF, _NBLK):
        out_copy(j).wait()


@jax.jit
def _run(inputs, W1, b1, W2, b2, W3, b3):
    n, k = inputs.shape
    h = W1.shape[1]
    o = W3.shape[1]
    const_spec = lambda shape: pl.BlockSpec(shape, lambda: (0, 0))
    return pl.pallas_call(
        _mlp_kernel,
        in_specs=[
            pl.BlockSpec(memory_space=pl.ANY),
            const_spec((k, h)),
            const_spec((1, h)),
            const_spec((h, h)),
            const_spec((1, h)),
            const_spec((h, o)),
            const_spec((1, o)),
        ],
        out_specs=pl.BlockSpec(memory_space=pl.ANY),
        out_shape=jax.ShapeDtypeStruct((n, o), jnp.float32),
        scratch_shapes=[
            pltpu.VMEM((_NBUF, _B, 1024), jnp.float32),
            pltpu.VMEM((_OBUF, _B, 64), jnp.float32),
            pltpu.SemaphoreType.DMA((_NBUF,)),
            pltpu.SemaphoreType.DMA((_OBUF,)),
        ],
    )(inputs, W1, b1.reshape(1, h), W2, b2.reshape(1, h), W3, b3.reshape(1, o))


def kernel(inputs, task_indices, W1, b1, W2, b2, W3, b3):
    del task_indices
    return _run(inputs, W1, b1, W2, b2, W3, b3)


# speedup vs baseline: 1.0548x; 1.0548x over previous
import jax
import jax.numpy as jnp
from jax.experimental import pallas as pl
from jax.experimental.pallas import tpu as pltpu

_N = 32768
_B = 4096
_NBLK = _N // _B
_NBUF = 3
_OBUF = 2


def _mlp_kernel(
    x_hbm,
    w1_ref,
    b1_ref,
    w2_ref,
    b2_ref,
    w3_ref,
    b3_ref,
    out_hbm,
    xbuf,
    obuf,
    insem,
    outsem,
):
    def in_copy(j):
        return pltpu.make_async_copy(
            x_hbm.at[pl.ds(j * _B, _B), :], xbuf.at[j % _NBUF], insem.at[j % _NBUF]
        )

    def out_copy(j):
        return pltpu.make_async_copy(
            obuf.at[j % _OBUF], out_hbm.at[pl.ds(j * _B, _B), :], outsem.at[j % _OBUF]
        )

    for j in range(_NBUF):
        in_copy(j).start()
    for i in range(_NBLK):
        in_copy(i).wait()
        h = jnp.tanh(
            jnp.dot(xbuf[i % _NBUF], w1_ref[...], preferred_element_type=jnp.float32)
            + b1_ref[...]
        )
        h = jnp.tanh(
            jnp.dot(h, w2_ref[...], preferred_element_type=jnp.float32) + b2_ref[...]
        )
        y = jnp.dot(h, w3_ref[...], preferred_element_type=jnp.float32) + b3_ref[...]
        if i >= _OBUF:
            out_copy(i - _OBUF).wait()
        obuf[i % _OBUF] = y
        out_copy(i).start()
        if i + _NBUF < _NBLK:
            in_copy(i + _NBUF).start()
    for j in range(_NBLK - _OBUF, _NBLK):
        out_copy(j).wait()


@jax.jit
def _run(inputs, W1, b1, W2, b2, W3, b3):
    n, k = inputs.shape
    h = W1.shape[1]
    o = W3.shape[1]
    const_spec = lambda shape: pl.BlockSpec(shape, lambda: (0, 0))
    return pl.pallas_call(
        _mlp_kernel,
        in_specs=[
            pl.BlockSpec(memory_space=pl.ANY),
            const_spec((k, h)),
            const_spec((1, h)),
            const_spec((h, h)),
            const_spec((1, h)),
            const_spec((h, o)),
            const_spec((1, o)),
        ],
        out_specs=pl.BlockSpec(memory_space=pl.ANY),
        out_shape=jax.ShapeDtypeStruct((n, o), jnp.float32),
        scratch_shapes=[
            pltpu.VMEM((_NBUF, _B, 1024), jnp.float32),
            pltpu.VMEM((_OBUF, _B, 64), jnp.float32),
            pltpu.SemaphoreType.DMA((_NBUF,)),
            pltpu.SemaphoreType.DMA((_OBUF,)),
        ],
    )(inputs, W1, b1.reshape(1, h), W2, b2.reshape(1, h), W3, b3.reshape(1, o))


def kernel(inputs, task_indices, W1, b1, W2, b2, W3, b3):
    del task_indices
    return _run(inputs, W1, b1, W2, b2, W3, b3)


# speedup vs baseline: 1.1162x; 1.0582x over previous
import jax
import jax.numpy as jnp
from jax.experimental import pallas as pl
from jax.experimental.pallas import tpu as pltpu

_BLOCK_ROWS = 4096


def _mlp_kernel(x_ref, w1_ref, b1_ref, w2_ref, b2_ref, w3_ref, b3_ref, out_ref):
    h = jnp.tanh(
        jnp.dot(x_ref[...], w1_ref[...], preferred_element_type=jnp.float32)
        + b1_ref[...]
    )
    h = jnp.tanh(
        jnp.dot(h, w2_ref[...], preferred_element_type=jnp.float32) + b2_ref[...]
    )
    out_ref[...] = (
        jnp.dot(h, w3_ref[...], preferred_element_type=jnp.float32) + b3_ref[...]
    )


@jax.jit
def _run(inputs, W1, b1, W2, b2, W3, b3):
    n, k = inputs.shape
    h = W1.shape[1]
    o = W3.shape[1]
    const_spec = lambda shape: pl.BlockSpec(shape, lambda i: (0, 0))
    return pl.pallas_call(
        _mlp_kernel,
        grid=(n // _BLOCK_ROWS,),
        in_specs=[
            pl.BlockSpec((_BLOCK_ROWS, k), lambda i: (i, 0)),
            const_spec((k, h)),
            const_spec((1, h)),
            const_spec((h, h)),
            const_spec((1, h)),
            const_spec((h, o)),
            const_spec((1, o)),
        ],
        out_specs=pl.BlockSpec((_BLOCK_ROWS, o), lambda i: (i, 0)),
        out_shape=jax.ShapeDtypeStruct((n, o), jnp.float32),
        compiler_params=pltpu.CompilerParams(
            dimension_semantics=("parallel",),
        ),
    )(inputs, W1, b1.reshape(1, h), W2, b2.reshape(1, h), W3, b3.reshape(1, o))


def kernel(inputs, task_indices, W1, b1, W2, b2, W3, b3):
    del task_indices
    return _run(inputs, W1, b1, W2, b2, W3, b3)
